# baseline (speedup 1.0000x reference)
import functools

import jax
import jax.numpy as jnp
from jax import lax
from jax.experimental import pallas as pl
from jax.experimental.pallas import tpu as pltpu
from jax.experimental.pallas import tpu_sc as plsc

N = 2048
D = 256
C = 3
CH_STRIDE = D * D * D

NC, NS, L = 2, 16, 16
NW = NC * NS
PER_W = N // NW
VECS = PER_W // L
SLICE = PER_W * C
HALF = SLICE // 2
CHUNKS = SLICE // L

_mesh = plsc.VectorSubcoreMesh(core_axis_name="c", subcore_axis_name="s",
                               num_cores=NC, num_subcores=NS)


@functools.partial(
    pl.kernel,
    mesh=_mesh,
    compiler_params=pltpu.CompilerParams(needs_layout_passes=False),
    out_type=jax.ShapeDtypeStruct((N * C,), jnp.float32),
    scratch_types=[
        pltpu.VMEM((SLICE,), jnp.int32),
        pltpu.VMEM((SLICE,), jnp.int32),
        pltpu.VMEM((SLICE,), jnp.float32),
        pltpu.VMEM((HALF,), jnp.int32),
        pltpu.VMEM((HALF,), jnp.int32),
        pltpu.VMEM((HALF,), jnp.float32),
        pltpu.VMEM((HALF,), jnp.float32),
        pltpu.VMEM((SLICE,), jnp.float32),
        pltpu.SemaphoreType.DMA,
    ],
)
def _trl_kernel(ml_hbm, fl_hbm, field_hbm, spac_hbm, out_hbm,
                ml_v, fl_v, sp_v, ia_v, ib_v, da_v, db_v, out_v, sem):
    wid = lax.axis_index("s") * NC + lax.axis_index("c")
    base = wid * SLICE

    pltpu.sync_copy(ml_hbm.at[pl.ds(base, SLICE)], ml_v)
    pltpu.sync_copy(fl_hbm.at[pl.ds(base, SLICE)], fl_v)
    pltpu.sync_copy(spac_hbm, sp_v)

    lane = lax.iota(jnp.int32, L)

    copies = []
    for half, tgt in ((0, ia_v), (1, ib_v)):
        for jj in range(VECS // 2):
            j = half * (VECS // 2) + jj
            nl3 = (j * L) * C + lane * C
            x = plsc.load_gather(ml_v, [nl3])
            y = plsc.load_gather(ml_v, [nl3 + 1])
            z = plsc.load_gather(ml_v, [nl3 + 2])
            lin = x * (D * D) + y * D + z
            pos = nl3 - half * HALF
            plsc.store_scatter(tgt, [pos], lin)
            plsc.store_scatter(tgt, [pos + 1], lin + CH_STRIDE)
            plsc.store_scatter(tgt, [pos + 2], lin + 2 * CH_STRIDE)
    cpa = pltpu.async_copy(field_hbm.at[ia_v], da_v, sem)
    cpb = pltpu.async_copy(field_hbm.at[ib_v], db_v, sem)

    cpa.wait()
    cpb.wait()

    for k in range(CHUNKS):
        sl = pl.ds(k * L, L)
        dv = da_v[pl.ds(k * L, L)] if k < CHUNKS // 2 else \
             db_v[pl.ds(k * L - HALF, L)]
        diff = (ml_v[sl] - fl_v[sl]).astype(jnp.float32)
        out_v[sl] = (diff + dv) * sp_v[sl]

    pltpu.sync_copy(out_v, out_hbm.at[pl.ds(base, SLICE)])


def kernel(fixed_landmarks, moving_landmarks, displacement_field,
           fixed_spacing, moving_spacing):
    del fixed_spacing
    ml_flat = moving_landmarks.reshape(-1)
    fl_flat = fixed_landmarks.reshape(-1)
    field_flat = displacement_field.reshape(-1)
    spac_tiled = jnp.tile(moving_spacing.astype(jnp.float32), SLICE // C)
    out_flat = _trl_kernel(ml_flat, fl_flat, field_flat, spac_tiled)
    return out_flat.reshape(N, C)


# speedup vs baseline: 5.8036x; 5.8036x over previous
import functools

import jax
import jax.numpy as jnp
from jax import lax
from jax.experimental import pallas as pl
from jax.experimental.pallas import tpu as pltpu
from jax.experimental.pallas import tpu_sc as plsc

N = 2048
D = 256
C = 3
CH_STRIDE = D * D * D

NC, NS, L = 2, 16, 16
NW = NC * NS
PER_W = N // NW
VECS = PER_W // L
SLICE = PER_W * C
HALF = SLICE // 2
CHUNKS = SLICE // L

_mesh = plsc.VectorSubcoreMesh(core_axis_name="c", subcore_axis_name="s",
                               num_cores=NC, num_subcores=NS)


@functools.partial(
    pl.kernel,
    mesh=_mesh,
    compiler_params=pltpu.CompilerParams(needs_layout_passes=False),
    out_type=jax.ShapeDtypeStruct((N * C,), jnp.float32),
    scratch_types=[
        pltpu.VMEM((SLICE,), jnp.int32),
        pltpu.VMEM((SLICE,), jnp.int32),
        pltpu.VMEM((SLICE,), jnp.float32),
        pltpu.VMEM((HALF,), jnp.int32),
        pltpu.VMEM((HALF,), jnp.int32),
        pltpu.VMEM((HALF,), jnp.float32),
        pltpu.VMEM((HALF,), jnp.float32),
        pltpu.VMEM((SLICE,), jnp.float32),
        pltpu.SemaphoreType.DMA,
    ],
)
def _trl_kernel(ml_hbm, fl_hbm, field_hbm, spac_hbm, out_hbm,
                ml_v, fl_v, sp_v, ia_v, ib_v, da_v, db_v, out_v, sem):
    wid = lax.axis_index("s") * NC + lax.axis_index("c")
    base = wid * SLICE

    pltpu.sync_copy(ml_hbm.at[pl.ds(base, SLICE)], ml_v)
    pltpu.sync_copy(fl_hbm.at[pl.ds(base, SLICE)], fl_v)
    pltpu.sync_copy(spac_hbm, sp_v)

    lane = lax.iota(jnp.int32, L)

    copies = []
    for half, tgt in ((0, ia_v), (1, ib_v)):
        for jj in range(VECS // 2):
            j = half * (VECS // 2) + jj
            nl3 = (j * L) * C + lane * C
            x = plsc.load_gather(ml_v, [nl3])
            y = plsc.load_gather(ml_v, [nl3 + 1])
            z = plsc.load_gather(ml_v, [nl3 + 2])
            tile = ((y >> 3) * 2 + (z >> 7)) * 1024
            lin = x * (D * D) + tile + (y & 7) * 128 + (z & 127)
            pos = nl3 - half * HALF
            plsc.store_scatter(tgt, [pos], lin)
            plsc.store_scatter(tgt, [pos + 1], lin + CH_STRIDE)
            plsc.store_scatter(tgt, [pos + 2], lin + 2 * CH_STRIDE)
    cpa = pltpu.async_copy(field_hbm.at[ia_v], da_v, sem)
    cpb = pltpu.async_copy(field_hbm.at[ib_v], db_v, sem)

    cpa.wait()
    cpb.wait()

    for k in range(CHUNKS):
        sl = pl.ds(k * L, L)
        dv = da_v[pl.ds(k * L, L)] if k < CHUNKS // 2 else \
             db_v[pl.ds(k * L - HALF, L)]
        diff = (ml_v[sl] - fl_v[sl]).astype(jnp.float32)
        out_v[sl] = (diff + dv) * sp_v[sl]

    pltpu.sync_copy(out_v, out_hbm.at[pl.ds(base, SLICE)])


def kernel(fixed_landmarks, moving_landmarks, displacement_field,
           fixed_spacing, moving_spacing):
    del fixed_spacing
    ml_flat = moving_landmarks.reshape(-1)
    fl_flat = fixed_landmarks.reshape(-1)
    field_flat = (displacement_field
                  .reshape(C, D, D // 8, 8, D // 128, 128)
                  .transpose(0, 1, 2, 4, 3, 5)
                  .reshape(-1))
    spac_tiled = jnp.tile(moving_spacing.astype(jnp.float32), SLICE // C)
    out_flat = _trl_kernel(ml_flat, fl_flat, field_flat, spac_tiled)
    return out_flat.reshape(N, C)


# speedup vs baseline: 6.4528x; 1.1119x over previous
import functools

import jax
import jax.numpy as jnp
from jax import lax
from jax.experimental import pallas as pl
from jax.experimental.pallas import tpu as pltpu
from jax.experimental.pallas import tpu_sc as plsc

N = 2048
D = 256
C = 3
CH_STRIDE = D * D * D

NC, NS, L = 2, 16, 16
NW = NC * NS
PER_W = N // NW
VECS = PER_W // L
SLICE = PER_W * C
HALF = SLICE // 2
CHUNKS = SLICE // L

_mesh = plsc.VectorSubcoreMesh(core_axis_name="c", subcore_axis_name="s",
                               num_cores=NC, num_subcores=NS)


@functools.partial(
    pl.kernel,
    mesh=_mesh,
    compiler_params=pltpu.CompilerParams(needs_layout_passes=False),
    out_type=jax.ShapeDtypeStruct((N * C,), jnp.float32),
    scratch_types=[
        pltpu.VMEM((SLICE,), jnp.int32),
        pltpu.VMEM((SLICE,), jnp.int32),
        pltpu.VMEM((C,), jnp.float32),
        pltpu.VMEM((HALF,), jnp.int32),
        pltpu.VMEM((HALF,), jnp.int32),
        pltpu.VMEM((HALF,), jnp.float32),
        pltpu.VMEM((HALF,), jnp.float32),
        pltpu.VMEM((SLICE,), jnp.float32),
        pltpu.SemaphoreType.DMA,
        pltpu.SemaphoreType.DMA,
        pltpu.SemaphoreType.DMA,
        pltpu.SemaphoreType.DMA,
    ],
)
def _trl_kernel(ml_hbm, fl_hbm, field_hbm, spac_hbm, out_hbm,
                ml_v, fl_v, ms_v, ia_v, ib_v, da_v, db_v, out_v,
                sem_g, sem_m, sem_f, sem_s):
    wid = lax.axis_index("s") * NC + lax.axis_index("c")
    base = wid * SLICE

    cpm = pltpu.async_copy(ml_hbm.at[pl.ds(base, SLICE)], ml_v, sem_m)
    cpf = pltpu.async_copy(fl_hbm.at[pl.ds(base, SLICE)], fl_v, sem_f)
    cps = pltpu.async_copy(spac_hbm, ms_v, sem_s)

    lane = lax.iota(jnp.int32, L)

    cpm.wait()
    copies = []
    for half, (tgt, dst) in ((0, (ia_v, da_v)), (1, (ib_v, db_v))):
        for jj in range(VECS // 2):
            j = half * (VECS // 2) + jj
            nl3 = (j * L) * C + lane * C
            x = plsc.load_gather(ml_v, [nl3])
            y = plsc.load_gather(ml_v, [nl3 + 1])
            z = plsc.load_gather(ml_v, [nl3 + 2])
            tile = ((y >> 3) * 2 + (z >> 7)) * 1024
            lin = x * (D * D) + tile + (y & 7) * 128 + (z & 127)
            pos = nl3 - half * HALF
            plsc.store_scatter(tgt, [pos], lin)
            plsc.store_scatter(tgt, [pos + 1], lin + CH_STRIDE)
            plsc.store_scatter(tgt, [pos + 2], lin + 2 * CH_STRIDE)
        copies.append(pltpu.async_copy(field_hbm.at[tgt], dst, sem_g))

    cps.wait()
    patt = [plsc.load_gather(ms_v, [lax.rem(lane + m, jnp.int32(C))])
            for m in range(C)]

    cpf.wait()
    for cp in copies:
        cp.wait()

    for k in range(CHUNKS):
        sl = pl.ds(k * L, L)
        dv = da_v[pl.ds(k * L, L)] if k < CHUNKS // 2 else \
             db_v[pl.ds(k * L - HALF, L)]
        diff = (ml_v[sl] - fl_v[sl]).astype(jnp.float32)
        out_v[sl] = (diff + dv) * patt[k % C]

    pltpu.sync_copy(out_v, out_hbm.at[pl.ds(base, SLICE)])


def kernel(fixed_landmarks, moving_landmarks, displacement_field,
           fixed_spacing, moving_spacing):
    del fixed_spacing
    ml_flat = moving_landmarks.reshape(-1)
    fl_flat = fixed_landmarks.reshape(-1)
    field_flat = (displacement_field
                  .reshape(C, D, D // 8, 8, D // 128, 128)
                  .transpose(0, 1, 2, 4, 3, 5)
                  .reshape(-1))
    out_flat = _trl_kernel(ml_flat, fl_flat, field_flat,
                           moving_spacing.astype(jnp.float32))
    return out_flat.reshape(N, C)


# speedup vs baseline: 7.9000x; 1.2243x over previous
import functools

import jax
import jax.numpy as jnp
from jax import lax
from jax.experimental import pallas as pl
from jax.experimental.pallas import tpu as pltpu
from jax.experimental.pallas import tpu_sc as plsc

N = 2048
D = 256
C = 3
CH_STRIDE = D * D * D

NC, NS, L = 2, 16, 16
NW = NC * NS
PER_W = N // NW
VECS = PER_W // L

_mesh = plsc.VectorSubcoreMesh(core_axis_name="c", subcore_axis_name="s",
                               num_cores=NC, num_subcores=NS)


@functools.partial(
    pl.kernel,
    mesh=_mesh,
    compiler_params=pltpu.CompilerParams(needs_layout_passes=False),
    out_type=jax.ShapeDtypeStruct((C, N), jnp.float32),
    scratch_types=[
        pltpu.VMEM((PER_W,), jnp.int32),
        pltpu.VMEM((PER_W,), jnp.int32),
        pltpu.VMEM((PER_W,), jnp.int32),
        pltpu.VMEM((PER_W,), jnp.int32),
        pltpu.VMEM((PER_W,), jnp.int32),
        pltpu.VMEM((PER_W,), jnp.int32),
        pltpu.VMEM((L,), jnp.float32),
        pltpu.VMEM((PER_W,), jnp.int32),
        pltpu.VMEM((PER_W,), jnp.int32),
        pltpu.VMEM((PER_W,), jnp.int32),
        pltpu.VMEM((PER_W,), jnp.float32),
        pltpu.VMEM((PER_W,), jnp.float32),
        pltpu.VMEM((PER_W,), jnp.float32),
        pltpu.VMEM((C, PER_W), jnp.float32),
        pltpu.SemaphoreType.DMA,
        pltpu.SemaphoreType.DMA,
        pltpu.SemaphoreType.DMA,
    ],
)
def _trl_kernel(mlT_hbm, flT_hbm, field_hbm, spac_hbm, out_hbm,
                xm_v, ym_v, zm_v, xf_v, yf_v, zf_v, ms_v,
                i0_v, i1_v, i2_v, d0_v, d1_v, d2_v, ob_v,
                sem_g, sem_m, sem_f):
    wid = lax.axis_index("s") * NC + lax.axis_index("c")
    n0 = wid * PER_W

    cpm = [pltpu.async_copy(mlT_hbm.at[c, pl.ds(n0, PER_W)], dst, sem_m)
           for c, dst in enumerate((xm_v, ym_v, zm_v))]
    cpf = [pltpu.async_copy(flT_hbm.at[c, pl.ds(n0, PER_W)], dst, sem_f)
           for c, dst in enumerate((xf_v, yf_v, zf_v))]
    cps = pltpu.async_copy(spac_hbm, ms_v.at[pl.ds(0, C)], sem_f)

    for cp in cpm:
        cp.wait()
    for j in range(VECS):
        sl = pl.ds(j * L, L)
        x, y, z = xm_v[sl], ym_v[sl], zm_v[sl]
        tile = ((y >> 3) * 2 + (z >> 7)) * 1024
        lin = x * (D * D) + tile + (y & 7) * 128 + (z & 127)
        i0_v[sl] = lin
        i1_v[sl] = lin + CH_STRIDE
        i2_v[sl] = lin + 2 * CH_STRIDE

    gathers = [pltpu.async_copy(field_hbm.at[iv], dv, sem_g)
               for iv, dv in ((i0_v, d0_v), (i1_v, d1_v), (i2_v, d2_v))]

    cps.wait()
    spv = ms_v[...]
    for cp in cpf:
        cp.wait()
    for cp in gathers:
        cp.wait()

    for c, (mv, fv, dv) in enumerate(((xm_v, xf_v, d0_v),
                                      (ym_v, yf_v, d1_v),
                                      (zm_v, zf_v, d2_v))):
        for j in range(VECS):
            sl = pl.ds(j * L, L)
            diff = (mv[sl] - fv[sl]).astype(jnp.float32)
            ob_v[c, sl] = (diff + dv[sl]) * spv[c]

    for c in range(C):
        pltpu.sync_copy(ob_v.at[c], out_hbm.at[c, pl.ds(n0, PER_W)])


def kernel(fixed_landmarks, moving_landmarks, displacement_field,
           fixed_spacing, moving_spacing):
    del fixed_spacing
    field_flat = (displacement_field
                  .reshape(C, D, D // 8, 8, D // 128, 128)
                  .transpose(0, 1, 2, 4, 3, 5)
                  .reshape(-1))
    outT = _trl_kernel(moving_landmarks.T, fixed_landmarks.T, field_flat,
                       moving_spacing.astype(jnp.float32))
    return outT.T
